# baseline (device time: 16745 ns/iter reference)
import os

import jax
import jax.numpy as jnp
from jax import lax
from jax.experimental import pallas as pl
from jax.experimental.pallas import tpu as pltpu

_ABLATE = os.environ.get("ABLATE", "none")

N_DEV = 4
B = 2
SQ = 128
H_LOC = 4
DH = 64
CHUNK = H_LOC * DH


def kernel(x, Wq, K_ext, V_ext, Wo):
    d_model = x.shape[-1]
    K_ext = K_ext.reshape(B, SQ, CHUNK)
    V_ext = V_ext.reshape(B, SQ, CHUNK)

    def body(x_ref, wq_ref, k_ref, v_ref, wo_ref, out_ref,
             comm_ref, send_sems, recv_sems):
        my_pos = lax.axis_index("i")

        if _ABLATE != "nocomm":
            barrier_sem = pltpu.get_barrier_semaphore()
            for j in range(1, N_DEV):
                pl.semaphore_signal(
                    barrier_sem, inc=1,
                    device_id=(lax.rem(my_pos + j, N_DEV),),
                    device_id_type=pl.DeviceIdType.MESH,
                )
            pl.semaphore_wait(barrier_sem, N_DEV - 1)

        if _ABLATE == "noattn":
            comm_ref[my_pos, :, :, :] = x_ref[:, :, :CHUNK].astype(jnp.bfloat16)
        else:
            xf = jnp.reshape(
                x_ref[:, :, :], (B * SQ, d_model)).astype(jnp.bfloat16)
            wq = wq_ref[:, pl.ds(my_pos * CHUNK, CHUNK)].astype(jnp.bfloat16)
            q = jnp.dot(xf, wq, preferred_element_type=jnp.float32)
            q = q.astype(jnp.bfloat16)

            blocks = []
            for b in range(B):
                kb = k_ref[b, :, :].astype(jnp.bfloat16)
                for h in range(H_LOC):
                    qh = q[b * SQ:(b + 1) * SQ, h * DH:(h + 1) * DH]
                    kh = kb[:, h * DH:(h + 1) * DH]
                    blocks.append(lax.dot_general(
                        qh, kh, (((1,), (1,)), ((), ())),
                        preferred_element_type=jnp.float32,
                    ))
            s = jnp.concatenate(blocks, axis=0) * 0.125
            s = s - jnp.max(s, axis=-1, keepdims=True)
            w = jnp.exp(s)
            w = (w / jnp.sum(w, axis=-1, keepdims=True)).astype(jnp.bfloat16)

            for b in range(B):
                vb = v_ref[b, :, :].astype(jnp.bfloat16)
                for h in range(H_LOC):
                    i = b * H_LOC + h
                    ctx = jnp.dot(
                        w[i * SQ:(i + 1) * SQ, :],
                        vb[:, h * DH:(h + 1) * DH],
                        preferred_element_type=jnp.float32,
                    )
                    comm_ref[my_pos, b, :, h * DH:(h + 1) * DH] = (
                        ctx.astype(jnp.bfloat16))

        sends = []
        if _ABLATE != "nocomm":
            for j in range(1, N_DEV):
                r = pltpu.make_async_remote_copy(
                    src_ref=comm_ref.at[my_pos],
                    dst_ref=comm_ref.at[my_pos],
                    send_sem=send_sems.at[j - 1],
                    recv_sem=recv_sems.at[my_pos],
                    device_id=(lax.rem(my_pos + j, N_DEV),),
                    device_id_type=pl.DeviceIdType.MESH,
                )
                r.start()
                sends.append(r)

        wo16 = wo_ref[:, :].astype(jnp.bfloat16)
        acc = None
        for o in range(N_DEV):
            if _ABLATE != "nocomm":
                recv = pltpu.make_async_remote_copy(
                    src_ref=comm_ref.at[o],
                    dst_ref=comm_ref.at[o],
                    send_sem=send_sems.at[0],
                    recv_sem=recv_sems.at[o],
                    device_id=(my_pos,),
                    device_id_type=pl.DeviceIdType.MESH,
                )

                @pl.when(o != my_pos)
                def _():
                    recv.wait_recv()

            chunk = jnp.reshape(comm_ref[o, :, :, :], (B * SQ, CHUNK))
            part = jnp.dot(
                chunk, wo16[o * CHUNK:(o + 1) * CHUNK, :],
                preferred_element_type=jnp.float32,
            )
            acc = part if acc is None else acc + part

        out_ref[:, :, :] = jnp.reshape(acc, (B, SQ, d_model))

        for r in sends:
            r.wait_send()

    return pl.pallas_call(
        body,
        out_shape=jax.ShapeDtypeStruct((B, SQ, d_model), jnp.float32),
        in_specs=[pl.BlockSpec(memory_space=pltpu.VMEM)] * 5,
        out_specs=pl.BlockSpec(memory_space=pltpu.VMEM),
        scratch_shapes=[
            pltpu.VMEM((N_DEV, B, SQ, CHUNK), jnp.bfloat16),
            pltpu.SemaphoreType.DMA((N_DEV - 1,)),
            pltpu.SemaphoreType.DMA((N_DEV,)),
        ],
        compiler_params=pltpu.CompilerParams(
            collective_id=None if _ABLATE == "nocomm" else 0),
    )(x, Wq, K_ext, V_ext, Wo)
